# baseline (speedup 1.0000x reference)
import functools

import jax
import jax.numpy as jnp
from jax import lax
from jax.experimental import pallas as pl
from jax.experimental.pallas import tpu as pltpu
from jax.experimental.pallas import tpu_sc as plsc

_N = 8192
_NB = 4
_K = 16
_GAMMA = 2.0
_LOSS_WEIGHT = 1.0
_BLK = _N // _NB
_QT = 256
_TPB = _BLK // _QT
_TC_T = 7
_NW = 32
_WPB = _NW // _NB
_QPW = (_BLK - _TC_T * _QT) // _WPB
_NG = _QPW // 16
_L = 16


def _prep_body(xyzT_ref, tm_ref, outT_ref, cT_ref, oT_ref):
    x = xyzT_ref[0:1, :]
    y = xyzT_ref[1:2, :]
    z = xyzT_ref[2:3, :]
    for r in range(3):
        t0 = tm_ref[4 * r + 0:4 * r + 1, :]
        t1 = tm_ref[4 * r + 1:4 * r + 2, :]
        t2 = tm_ref[4 * r + 2:4 * r + 3, :]
        t3 = tm_ref[4 * r + 3:4 * r + 4, :]
        cT_ref[r:r + 1, :] = t0 * x + t1 * y + t2 * z + t3
    ox = outT_ref[0:1, :]
    oy = outT_ref[1:2, :]
    oz = outT_ref[2:3, :]
    denom = jnp.maximum(jnp.sqrt(ox * ox + oy * oy + oz * oz), 1e-12)
    ox = ox / denom
    oy = oy / denom
    oz = oz / denom
    for r in range(3):
        t0 = tm_ref[4 * r + 0:4 * r + 1, :]
        t1 = tm_ref[4 * r + 1:4 * r + 2, :]
        t2 = tm_ref[4 * r + 2:4 * r + 3, :]
        oT_ref[r:r + 1, :] = t0 * ox + t1 * oy + t2 * oz


def _main_body(qc_ref, qo_ref, ccT_ref, coT_ref, acc_ref):
    i = pl.program_id(0)

    @pl.when(i == 0)
    def _init():
        acc_ref[:, :] = jnp.zeros((1, 1), dtype=jnp.float32)

    qc = qc_ref[:, :]
    qn = jnp.sum(qc * qc, axis=1, keepdims=True)
    cx = ccT_ref[0:1, :]
    cy = ccT_ref[1:2, :]
    cz = ccT_ref[2:3, :]
    cn = cx * cx + cy * cy + cz * cz
    dot = jnp.dot(qc, ccT_ref[:, :],
                  preferred_element_type=jnp.float32)
    d2 = (qn + cn) - 2.0 * dot

    l1 = (jnp.abs(qo_ref[:, 0:1] - coT_ref[0:1, :])
          + jnp.abs(qo_ref[:, 1:2] - coT_ref[1:2, :])
          + jnp.abs(qo_ref[:, 2:3] - coT_ref[2:3, :]))

    t = jnp.full((_QT, 1), -jnp.inf, dtype=jnp.float32)
    for _ in range(_K):
        t = jnp.min(jnp.where(d2 > t, d2, jnp.inf), axis=1, keepdims=True)

    val = jnp.exp(d2 * (-1.0 / (_GAMMA * _GAMMA))) * l1
    lt = d2 < t
    eq = d2 == t
    n_lt = jnp.sum(lt.astype(jnp.float32), axis=1, keepdims=True)
    n_eq = jnp.sum(eq.astype(jnp.float32), axis=1, keepdims=True)
    s_lt = jnp.sum(jnp.where(lt, val, 0.0), axis=1, keepdims=True)
    s_eq = jnp.sum(jnp.where(eq, val, 0.0), axis=1, keepdims=True)
    factor = jnp.clip(_K - n_lt, 0.0, n_eq) / jnp.maximum(n_eq, 1.0)
    rows = s_lt + s_eq * factor
    part = jnp.sum(rows, keepdims=True).reshape(1, 1)
    acc_ref[:, :] = acc_ref[:, :] + part


def _splat_i(j):
    return jnp.full((_L,), j, dtype=jnp.int32)


def _sc_body(cx_h, cy_h, cz_h, ox_h, oy_h, oz_h, out_h,
             cxv, cyv, czv, oxv, oyv, ozv,
             qxv, qyv, qzv, qoxv, qoyv, qozv,
             d2v, accv):
    cid = lax.axis_index("c")
    sid = lax.axis_index("s")
    wid = sid * 2 + cid
    blk = wid // _WPB
    qbase = blk * _BLK + _TC_T * _QT + (wid % _WPB) * _QPW
    cand_base = (blk ^ 1) * _BLK

    pltpu.sync_copy(cx_h.at[pl.ds(cand_base, _BLK)], cxv)
    pltpu.sync_copy(cy_h.at[pl.ds(cand_base, _BLK)], cyv)
    pltpu.sync_copy(cz_h.at[pl.ds(cand_base, _BLK)], czv)
    pltpu.sync_copy(ox_h.at[pl.ds(cand_base, _BLK)], oxv)
    pltpu.sync_copy(oy_h.at[pl.ds(cand_base, _BLK)], oyv)
    pltpu.sync_copy(oz_h.at[pl.ds(cand_base, _BLK)], ozv)
    pltpu.sync_copy(cx_h.at[pl.ds(qbase, _QPW)], qxv)
    pltpu.sync_copy(cy_h.at[pl.ds(qbase, _QPW)], qyv)
    pltpu.sync_copy(cz_h.at[pl.ds(qbase, _QPW)], qzv)
    pltpu.sync_copy(ox_h.at[pl.ds(qbase, _QPW)], qoxv)
    pltpu.sync_copy(oy_h.at[pl.ds(qbase, _QPW)], qoyv)
    pltpu.sync_copy(oz_h.at[pl.ds(qbase, _QPW)], qozv)

    inf = jnp.full((_L,), jnp.inf, dtype=jnp.float32)
    zero = jnp.zeros((_L,), dtype=jnp.float32)
    _UNROLL = 8
    _NCH = _BLK // _UNROLL

    def group_body(g, acc):
        qx = qxv[pl.ds(g * _L, _L)]
        qy = qyv[pl.ds(g * _L, _L)]
        qz = qzv[pl.ds(g * _L, _L)]
        qox = qoxv[pl.ds(g * _L, _L)]
        qoy = qoyv[pl.ds(g * _L, _L)]
        qoz = qozv[pl.ds(g * _L, _L)]

        def p0(jo, mins):
            ms = list(mins)
            for u in range(_UNROLL):
                j = jo * _UNROLL + u
                idx = _splat_i(j)
                bx = plsc.load_gather(cxv, [idx])
                by = plsc.load_gather(cyv, [idx])
                bz = plsc.load_gather(czv, [idx])
                dx = qx - bx
                dy = qy - by
                dz = qz - bz
                d2 = dx * dx + dy * dy + dz * dz
                d2v[j] = d2
                ms[u % 4] = jnp.minimum(ms[u % 4], d2)
            return tuple(ms)

        m0, m1, m2, m3 = lax.fori_loop(0, _NCH, p0, (inf, inf, inf, inf))
        t = jnp.minimum(jnp.minimum(m0, m1), jnp.minimum(m2, m3))

        def p1_step(_, t):
            def p1(jo, mins):
                ms = list(mins)
                for u in range(_UNROLL):
                    j = jo * _UNROLL + u
                    v = d2v[j]
                    ms[u % 4] = jnp.minimum(
                        ms[u % 4], jnp.where(v > t, v, inf))
                return tuple(ms)
            m0, m1, m2, m3 = lax.fori_loop(0, _NCH, p1, (inf, inf, inf, inf))
            return jnp.minimum(jnp.minimum(m0, m1), jnp.minimum(m2, m3))

        t = lax.fori_loop(0, _K - 1, p1_step, t)

        def p2(jo, carry):
            s_lt, s_eq, n_lt, n_eq = carry
            for u in range(_UNROLL):
                j = jo * _UNROLL + u
                v = d2v[j]
                idx = _splat_i(j)
                bx = plsc.load_gather(oxv, [idx])
                by = plsc.load_gather(oyv, [idx])
                bz = plsc.load_gather(ozv, [idx])
                l1 = (jnp.abs(qox - bx) + jnp.abs(qoy - by)
                      + jnp.abs(qoz - bz))
                contrib = jnp.exp(v * (-1.0 / (_GAMMA * _GAMMA))) * l1
                is_lt = v < t
                is_eq = v == t
                s_lt = s_lt + jnp.where(is_lt, contrib, zero)
                s_eq = s_eq + jnp.where(is_eq, contrib, zero)
                n_lt = n_lt + jnp.where(is_lt, 1.0, 0.0)
                n_eq = n_eq + jnp.where(is_eq, 1.0, 0.0)
            return (s_lt, s_eq, n_lt, n_eq)

        s_lt, s_eq, n_lt, n_eq = lax.fori_loop(
            0, _NCH, p2, (zero, zero, zero, zero))
        factor = jnp.clip(_K - n_lt, 0.0, n_eq) / jnp.maximum(n_eq, 1.0)
        return acc + s_lt + s_eq * factor

    acc = lax.fori_loop(0, _NG, group_body, zero)
    accv[...] = acc
    pltpu.sync_copy(accv, out_h.at[wid])


def _sc_call(cx, cy, cz, ox, oy, oz):
    mesh = plsc.VectorSubcoreMesh(core_axis_name="c", subcore_axis_name="s")
    f = functools.partial(
        pl.kernel,
        mesh=mesh,
        compiler_params=pltpu.CompilerParams(
            needs_layout_passes=False, use_tc_tiling_on_sc=False),
        out_type=jax.ShapeDtypeStruct((_NW, _L), jnp.float32),
        scratch_types=[
            pltpu.VMEM((_BLK,), jnp.float32),
            pltpu.VMEM((_BLK,), jnp.float32),
            pltpu.VMEM((_BLK,), jnp.float32),
            pltpu.VMEM((_BLK,), jnp.float32),
            pltpu.VMEM((_BLK,), jnp.float32),
            pltpu.VMEM((_BLK,), jnp.float32),
            pltpu.VMEM((_QPW,), jnp.float32),
            pltpu.VMEM((_QPW,), jnp.float32),
            pltpu.VMEM((_QPW,), jnp.float32),
            pltpu.VMEM((_QPW,), jnp.float32),
            pltpu.VMEM((_QPW,), jnp.float32),
            pltpu.VMEM((_QPW,), jnp.float32),
            pltpu.VMEM((_BLK, _L), jnp.float32),
            pltpu.VMEM((_L,), jnp.float32),
        ],
    )(_sc_body)
    return f(cx, cy, cz, ox, oy, oz)


def kernel(coord, intensity, out, target, untransform_coord):
    del intensity, target
    xyzT = coord[:, 1:4].T.astype(jnp.float32)
    tm = untransform_coord.reshape(_N, 16).T.astype(jnp.float32)
    outT = out.T.astype(jnp.float32)

    cT, oT = pl.pallas_call(
        _prep_body,
        out_shape=[jax.ShapeDtypeStruct((3, _N), jnp.float32),
                   jax.ShapeDtypeStruct((3, _N), jnp.float32)],
    )(xyzT, tm, outT)

    c = cT.T
    o = oT.T

    parts_sc = _sc_call(cT[0], cT[1], cT[2], oT[0], oT[1], oT[2])

    nprog = _NB * _TC_T

    def _qrow(i):
        return (i // _TC_T) * _TPB + (i % _TC_T)

    acc = pl.pallas_call(
        _main_body,
        grid=(nprog,),
        in_specs=[
            pl.BlockSpec((_QT, 3), lambda i: (_qrow(i), 0)),
            pl.BlockSpec((_QT, 3), lambda i: (_qrow(i), 0)),
            pl.BlockSpec((3, _BLK), lambda i: (0, (i // _TC_T) ^ 1)),
            pl.BlockSpec((3, _BLK), lambda i: (0, (i // _TC_T) ^ 1)),
        ],
        out_specs=pl.BlockSpec((1, 1), lambda i: (0, 0)),
        out_shape=jax.ShapeDtypeStruct((1, 1), jnp.float32),
    )(c, o, cT, oT)

    total = acc[0, 0] + jnp.sum(parts_sc)
    return (total * (_LOSS_WEIGHT / (_N * _K))).astype(jnp.float32)


# speedup vs baseline: 1.0196x; 1.0196x over previous
import functools

import jax
import jax.numpy as jnp
from jax import lax
from jax.experimental import pallas as pl
from jax.experimental.pallas import tpu as pltpu
from jax.experimental.pallas import tpu_sc as plsc

_N = 8192
_NB = 4
_K = 16
_GAMMA = 2.0
_LOSS_WEIGHT = 1.0
_BLK = _N // _NB
_QT = 256
_TPB = _BLK // _QT
_TC_T = 7
_NW = 32
_WPB = _NW // _NB
_QPW = (_BLK - _TC_T * _QT) // _WPB
_NG = _QPW // 16
_L = 16


def _prep_body(xyzT_ref, tm_ref, outT_ref, cT_ref, oT_ref):
    x = xyzT_ref[0:1, :]
    y = xyzT_ref[1:2, :]
    z = xyzT_ref[2:3, :]
    for r in range(3):
        t0 = tm_ref[4 * r + 0:4 * r + 1, :]
        t1 = tm_ref[4 * r + 1:4 * r + 2, :]
        t2 = tm_ref[4 * r + 2:4 * r + 3, :]
        t3 = tm_ref[4 * r + 3:4 * r + 4, :]
        cT_ref[r:r + 1, :] = t0 * x + t1 * y + t2 * z + t3
    ox = outT_ref[0:1, :]
    oy = outT_ref[1:2, :]
    oz = outT_ref[2:3, :]
    denom = jnp.maximum(jnp.sqrt(ox * ox + oy * oy + oz * oz), 1e-12)
    ox = ox / denom
    oy = oy / denom
    oz = oz / denom
    for r in range(3):
        t0 = tm_ref[4 * r + 0:4 * r + 1, :]
        t1 = tm_ref[4 * r + 1:4 * r + 2, :]
        t2 = tm_ref[4 * r + 2:4 * r + 3, :]
        oT_ref[r:r + 1, :] = t0 * ox + t1 * oy + t2 * oz


def _main_body(qcT_ref, qoT_ref, ccT_ref, coT_ref, acc_ref):
    i = pl.program_id(0)

    @pl.when(i == 0)
    def _init():
        acc_ref[:, :] = jnp.zeros((1, 1), dtype=jnp.float32)

    qc = jnp.transpose(qcT_ref[:, :], (1, 0))
    qo_t = jnp.transpose(qoT_ref[:, :], (1, 0))
    qn = jnp.sum(qc * qc, axis=1, keepdims=True)
    cx = ccT_ref[0:1, :]
    cy = ccT_ref[1:2, :]
    cz = ccT_ref[2:3, :]
    cn = cx * cx + cy * cy + cz * cz
    dot = jnp.dot(qc, ccT_ref[:, :],
                  preferred_element_type=jnp.float32)
    d2 = (qn + cn) - 2.0 * dot

    l1 = (jnp.abs(qo_t[:, 0:1] - coT_ref[0:1, :])
          + jnp.abs(qo_t[:, 1:2] - coT_ref[1:2, :])
          + jnp.abs(qo_t[:, 2:3] - coT_ref[2:3, :]))

    t = jnp.full((_QT, 1), -jnp.inf, dtype=jnp.float32)
    for _ in range(_K):
        t = jnp.min(jnp.where(d2 > t, d2, jnp.inf), axis=1, keepdims=True)

    val = jnp.exp(d2 * (-1.0 / (_GAMMA * _GAMMA))) * l1
    lt = d2 < t
    eq = d2 == t
    n_lt = jnp.sum(lt.astype(jnp.float32), axis=1, keepdims=True)
    n_eq = jnp.sum(eq.astype(jnp.float32), axis=1, keepdims=True)
    s_lt = jnp.sum(jnp.where(lt, val, 0.0), axis=1, keepdims=True)
    s_eq = jnp.sum(jnp.where(eq, val, 0.0), axis=1, keepdims=True)
    factor = jnp.clip(_K - n_lt, 0.0, n_eq) / jnp.maximum(n_eq, 1.0)
    rows = s_lt + s_eq * factor
    part = jnp.sum(rows, keepdims=True).reshape(1, 1)
    acc_ref[:, :] = acc_ref[:, :] + part


def _splat_i(j):
    return jnp.full((_L,), j, dtype=jnp.int32)


def _sc_body(cT_h, oT_h, out_h,
             cxv, cyv, czv, oxv, oyv, ozv,
             qxv, qyv, qzv, qoxv, qoyv, qozv,
             d2v, accv):
    cid = lax.axis_index("c")
    sid = lax.axis_index("s")
    wid = sid * 2 + cid
    blk = wid // _WPB
    qbase = blk * _BLK + _TC_T * _QT + (wid % _WPB) * _QPW
    cand_base = (blk ^ 1) * _BLK

    pltpu.sync_copy(cT_h.at[0, pl.ds(cand_base, _BLK)], cxv)
    pltpu.sync_copy(cT_h.at[1, pl.ds(cand_base, _BLK)], cyv)
    pltpu.sync_copy(cT_h.at[2, pl.ds(cand_base, _BLK)], czv)
    pltpu.sync_copy(oT_h.at[0, pl.ds(cand_base, _BLK)], oxv)
    pltpu.sync_copy(oT_h.at[1, pl.ds(cand_base, _BLK)], oyv)
    pltpu.sync_copy(oT_h.at[2, pl.ds(cand_base, _BLK)], ozv)
    pltpu.sync_copy(cT_h.at[0, pl.ds(qbase, _QPW)], qxv)
    pltpu.sync_copy(cT_h.at[1, pl.ds(qbase, _QPW)], qyv)
    pltpu.sync_copy(cT_h.at[2, pl.ds(qbase, _QPW)], qzv)
    pltpu.sync_copy(oT_h.at[0, pl.ds(qbase, _QPW)], qoxv)
    pltpu.sync_copy(oT_h.at[1, pl.ds(qbase, _QPW)], qoyv)
    pltpu.sync_copy(oT_h.at[2, pl.ds(qbase, _QPW)], qozv)

    inf = jnp.full((_L,), jnp.inf, dtype=jnp.float32)
    zero = jnp.zeros((_L,), dtype=jnp.float32)
    _UNROLL = 8
    _NCH = _BLK // _UNROLL

    def group_body(g, acc):
        qx = qxv[pl.ds(g * _L, _L)]
        qy = qyv[pl.ds(g * _L, _L)]
        qz = qzv[pl.ds(g * _L, _L)]
        qox = qoxv[pl.ds(g * _L, _L)]
        qoy = qoyv[pl.ds(g * _L, _L)]
        qoz = qozv[pl.ds(g * _L, _L)]

        def p0(jo, mins):
            ms = list(mins)
            for u in range(_UNROLL):
                j = jo * _UNROLL + u
                idx = _splat_i(j)
                bx = plsc.load_gather(cxv, [idx])
                by = plsc.load_gather(cyv, [idx])
                bz = plsc.load_gather(czv, [idx])
                dx = qx - bx
                dy = qy - by
                dz = qz - bz
                d2 = dx * dx + dy * dy + dz * dz
                d2v[j] = d2
                ms[u % 4] = jnp.minimum(ms[u % 4], d2)
            return tuple(ms)

        m0, m1, m2, m3 = lax.fori_loop(0, _NCH, p0, (inf, inf, inf, inf))
        t = jnp.minimum(jnp.minimum(m0, m1), jnp.minimum(m2, m3))

        def p1_step(_, t):
            def p1(jo, mins):
                ms = list(mins)
                for u in range(_UNROLL):
                    j = jo * _UNROLL + u
                    v = d2v[j]
                    ms[u % 4] = jnp.minimum(
                        ms[u % 4], jnp.where(v > t, v, inf))
                return tuple(ms)
            m0, m1, m2, m3 = lax.fori_loop(0, _NCH, p1, (inf, inf, inf, inf))
            return jnp.minimum(jnp.minimum(m0, m1), jnp.minimum(m2, m3))

        t = lax.fori_loop(0, _K - 1, p1_step, t)

        def p2(jo, carry):
            s_lt, s_eq, n_lt, n_eq = carry
            for u in range(_UNROLL):
                j = jo * _UNROLL + u
                v = d2v[j]
                idx = _splat_i(j)
                bx = plsc.load_gather(oxv, [idx])
                by = plsc.load_gather(oyv, [idx])
                bz = plsc.load_gather(ozv, [idx])
                l1 = (jnp.abs(qox - bx) + jnp.abs(qoy - by)
                      + jnp.abs(qoz - bz))
                contrib = jnp.exp(v * (-1.0 / (_GAMMA * _GAMMA))) * l1
                is_lt = v < t
                is_eq = v == t
                s_lt = s_lt + jnp.where(is_lt, contrib, zero)
                s_eq = s_eq + jnp.where(is_eq, contrib, zero)
                n_lt = n_lt + jnp.where(is_lt, 1.0, 0.0)
                n_eq = n_eq + jnp.where(is_eq, 1.0, 0.0)
            return (s_lt, s_eq, n_lt, n_eq)

        s_lt, s_eq, n_lt, n_eq = lax.fori_loop(
            0, _NCH, p2, (zero, zero, zero, zero))
        factor = jnp.clip(_K - n_lt, 0.0, n_eq) / jnp.maximum(n_eq, 1.0)
        return acc + s_lt + s_eq * factor

    acc = lax.fori_loop(0, _NG, group_body, zero)
    accv[...] = acc
    pltpu.sync_copy(accv, out_h.at[wid])


def _sc_call(cT, oT):
    mesh = plsc.VectorSubcoreMesh(core_axis_name="c", subcore_axis_name="s")
    f = functools.partial(
        pl.kernel,
        mesh=mesh,
        compiler_params=pltpu.CompilerParams(
            needs_layout_passes=False, use_tc_tiling_on_sc=False),
        out_type=jax.ShapeDtypeStruct((_NW, _L), jnp.float32),
        scratch_types=[
            pltpu.VMEM((_BLK,), jnp.float32),
            pltpu.VMEM((_BLK,), jnp.float32),
            pltpu.VMEM((_BLK,), jnp.float32),
            pltpu.VMEM((_BLK,), jnp.float32),
            pltpu.VMEM((_BLK,), jnp.float32),
            pltpu.VMEM((_BLK,), jnp.float32),
            pltpu.VMEM((_QPW,), jnp.float32),
            pltpu.VMEM((_QPW,), jnp.float32),
            pltpu.VMEM((_QPW,), jnp.float32),
            pltpu.VMEM((_QPW,), jnp.float32),
            pltpu.VMEM((_QPW,), jnp.float32),
            pltpu.VMEM((_QPW,), jnp.float32),
            pltpu.VMEM((_BLK, _L), jnp.float32),
            pltpu.VMEM((_L,), jnp.float32),
        ],
    )(_sc_body)
    return f(cT, oT)


def kernel(coord, intensity, out, target, untransform_coord):
    del intensity, target
    xyzT = coord[:, 1:4].T.astype(jnp.float32)
    tm = untransform_coord.reshape(_N, 16).T.astype(jnp.float32)
    outT = out.T.astype(jnp.float32)

    cT, oT = pl.pallas_call(
        _prep_body,
        out_shape=[jax.ShapeDtypeStruct((3, _N), jnp.float32),
                   jax.ShapeDtypeStruct((3, _N), jnp.float32)],
    )(xyzT, tm, outT)

    parts_sc = _sc_call(cT, oT)

    nprog = _NB * _TC_T

    def _qrow(i):
        return (i // _TC_T) * _TPB + (i % _TC_T)

    acc = pl.pallas_call(
        _main_body,
        grid=(nprog,),
        in_specs=[
            pl.BlockSpec((3, _QT), lambda i: (0, _qrow(i))),
            pl.BlockSpec((3, _QT), lambda i: (0, _qrow(i))),
            pl.BlockSpec((3, _BLK), lambda i: (0, (i // _TC_T) ^ 1)),
            pl.BlockSpec((3, _BLK), lambda i: (0, (i // _TC_T) ^ 1)),
        ],
        out_specs=pl.BlockSpec((1, 1), lambda i: (0, 0)),
        out_shape=jax.ShapeDtypeStruct((1, 1), jnp.float32),
    )(cT, oT, cT, oT)

    total = acc[0, 0] + jnp.sum(parts_sc)
    return (total * (_LOSS_WEIGHT / (_N * _K))).astype(jnp.float32)
